# Initial kernel scaffold; baseline (speedup 1.0000x reference)
import jax
import jax.numpy as jnp
from jax.experimental import pallas as pl


def kernel(e1, rel, X, A, ent_emb, rel_emb):
    raise NotImplementedError("write your pallas kernel here")


# speedup vs baseline: 1.0478x; 1.0478x over previous
import functools

import jax
import jax.numpy as jnp
from jax import lax
from jax.experimental import pallas as pl
from jax.experimental.pallas import tpu as pltpu
from jax.experimental.pallas import tpu_sc as plsc

B = 1024
NUM_ENT = 1000
DIM = 64
D_PAD = 128
N_PAD = 1024
B_BLK = 128
CH = 128
EPS = 1e-12


def _sc_gather(ent_emb, rel_emb, e1, rel):
    info = plsc.get_sparse_core_info()
    nw = info.num_cores * info.num_subcores
    b_per_w = B // nw
    mesh = plsc.VectorSubcoreMesh(core_axis_name="c", subcore_axis_name="s")

    @functools.partial(
        pl.kernel,
        mesh=mesh,
        out_type=[
            jax.ShapeDtypeStruct((B, D_PAD), jnp.float32),
            jax.ShapeDtypeStruct((B, D_PAD), jnp.float32),
        ],
        scratch_types=[
            pltpu.VMEM((b_per_w,), jnp.int32),
            pltpu.VMEM((b_per_w,), jnp.int32),
            pltpu.VMEM((b_per_w, D_PAD), jnp.float32),
            pltpu.VMEM((b_per_w, D_PAD), jnp.float32),
            pltpu.SemaphoreType.DMA,
            pltpu.SemaphoreType.DMA,
        ],
    )
    def gk(ent_hbm, rel_hbm, e1_hbm, ridx_hbm, oute_hbm, outr_hbm,
           idx1_v, idx2_v, rows1_v, rows2_v, sem1, sem2):
        wid = lax.axis_index("s") * info.num_cores + lax.axis_index("c")
        base = wid * b_per_w
        pltpu.sync_copy(e1_hbm.at[pl.ds(base, b_per_w)], idx1_v)
        pltpu.sync_copy(ridx_hbm.at[pl.ds(base, b_per_w)], idx2_v)
        c1 = pltpu.async_copy(ent_hbm.at[idx1_v], rows1_v, sem1)
        c2 = pltpu.async_copy(rel_hbm.at[idx2_v], rows2_v, sem2)
        c1.wait()
        c2.wait()
        pltpu.sync_copy(rows1_v, oute_hbm.at[pl.ds(base, b_per_w)])
        pltpu.sync_copy(rows2_v, outr_hbm.at[pl.ds(base, b_per_w)])

    return gk(ent_emb, rel_emb, e1, rel)


def _score_body(e1r_ref, relr_ref, entT_ref, out_ref):
    def rnorm(x):
        n = jnp.sqrt(jnp.sum(x * x, axis=-1, keepdims=True))
        return x / jnp.maximum(n, EPS)

    h = rnorm(e1r_ref[:, :DIM]) + rnorm(relr_ref[:, :DIM])

    ent_t = entT_ref[...]
    n = jnp.sqrt(jnp.sum(ent_t * ent_t, axis=0, keepdims=True))
    ent_tn = ent_t / jnp.maximum(n, EPS)

    chunks = []
    for c in range(N_PAD // CH):
        sl = ent_tn[:, c * CH:(c + 1) * CH]
        acc = jnp.abs(h[:, 0:1] - sl[0:1, :])
        for d in range(1, DIM):
            acc = acc + jnp.abs(h[:, d:d + 1] - sl[d:d + 1, :])
        chunks.append(acc)
    dist = jnp.concatenate(chunks, axis=1)

    lane = lax.broadcasted_iota(jnp.int32, (1, N_PAD), 1)
    valid = lane < NUM_ENT
    dist = jnp.where(valid, dist, -jnp.inf)
    m = jnp.max(dist, axis=-1, keepdims=True)
    e = jnp.exp(dist - m)
    e = jnp.where(valid, e, 0.0)
    s = jnp.sum(e, axis=-1, keepdims=True)
    out_ref[...] = (e / s)[:, :NUM_ENT]


def kernel(e1, rel, X, A, ent_emb, rel_emb):
    del X, A
    e1 = e1.astype(jnp.int32)
    rel = rel.astype(jnp.int32)
    ent_pad = jnp.pad(ent_emb, ((0, 0), (0, D_PAD - DIM)))
    rel_pad = jnp.pad(rel_emb, ((0, 0), (0, D_PAD - DIM)))
    e1_rows, rel_rows = _sc_gather(ent_pad, rel_pad, e1, rel)
    ent_t = jnp.pad(ent_emb.T, ((0, 0), (0, N_PAD - NUM_ENT)))
    return pl.pallas_call(
        _score_body,
        grid=(B // B_BLK,),
        in_specs=[
            pl.BlockSpec((B_BLK, D_PAD), lambda i: (i, 0)),
            pl.BlockSpec((B_BLK, D_PAD), lambda i: (i, 0)),
            pl.BlockSpec((DIM, N_PAD), lambda i: (0, 0)),
        ],
        out_specs=pl.BlockSpec((B_BLK, NUM_ENT), lambda i: (i, 0)),
        out_shape=jax.ShapeDtypeStruct((B, NUM_ENT), jnp.float32),
    )(e1_rows, rel_rows, ent_t)


# speedup vs baseline: 1.1552x; 1.1025x over previous
import functools

import jax
import jax.numpy as jnp
from jax import lax
from jax.experimental import pallas as pl
from jax.experimental.pallas import tpu as pltpu
from jax.experimental.pallas import tpu_sc as plsc

B = 1024
NUM_ENT = 1000
DIM = 64
D_PAD = 128
N_PAD = 1024
B_BLK = 128
CH = 128
EPS = 1e-12


def _sc_gather(ent_emb, rel_emb, e1, rel):
    info = plsc.get_sparse_core_info()
    nw = info.num_cores * info.num_subcores
    b_per_w = B // nw
    mesh = plsc.VectorSubcoreMesh(core_axis_name="c", subcore_axis_name="s")

    @functools.partial(
        pl.kernel,
        mesh=mesh,
        out_type=[
            jax.ShapeDtypeStruct((B, D_PAD), jnp.float32),
            jax.ShapeDtypeStruct((B, D_PAD), jnp.float32),
        ],
        scratch_types=[
            pltpu.VMEM((b_per_w,), jnp.int32),
            pltpu.VMEM((b_per_w,), jnp.int32),
            pltpu.VMEM((b_per_w, D_PAD), jnp.float32),
            pltpu.VMEM((b_per_w, D_PAD), jnp.float32),
            pltpu.SemaphoreType.DMA,
            pltpu.SemaphoreType.DMA,
        ],
    )
    def gk(ent_hbm, rel_hbm, e1_hbm, ridx_hbm, oute_hbm, outr_hbm,
           idx1_v, idx2_v, rows1_v, rows2_v, sem1, sem2):
        wid = lax.axis_index("s") * info.num_cores + lax.axis_index("c")
        base = wid * b_per_w
        pltpu.sync_copy(e1_hbm.at[pl.ds(base, b_per_w)], idx1_v)
        pltpu.sync_copy(ridx_hbm.at[pl.ds(base, b_per_w)], idx2_v)
        c1 = pltpu.async_copy(ent_hbm.at[idx1_v], rows1_v, sem1)
        c2 = pltpu.async_copy(rel_hbm.at[idx2_v], rows2_v, sem2)
        c1.wait()
        c2.wait()
        pltpu.sync_copy(rows1_v, oute_hbm.at[pl.ds(base, b_per_w)])
        pltpu.sync_copy(rows2_v, outr_hbm.at[pl.ds(base, b_per_w)])

    return gk(ent_emb, rel_emb, e1, rel)


def _score_body(e1r_ref, relr_ref, entT_ref, out_ref):
    def rnorm(x):
        n = jnp.sqrt(jnp.sum(x * x, axis=-1, keepdims=True))
        return x / jnp.maximum(n, EPS)

    h = rnorm(e1r_ref[:, :DIM]) + rnorm(relr_ref[:, :DIM])

    ent_t = entT_ref[...]
    n = jnp.sqrt(jnp.sum(ent_t * ent_t, axis=0, keepdims=True))
    ent_tn = ent_t / jnp.maximum(n, EPS)

    base = (jnp.sum(h, axis=-1, keepdims=True)
            + jnp.sum(ent_tn, axis=0, keepdims=True))
    hb = [jnp.broadcast_to(h[:, d:d + 1], (B_BLK, CH)) for d in range(DIM)]

    chunks = []
    for c in range(N_PAD // CH):
        sl = ent_tn[:, c * CH:(c + 1) * CH]
        acc = jnp.minimum(hb[0], sl[0:1, :])
        for d in range(1, DIM):
            acc = acc + jnp.minimum(hb[d], sl[d:d + 1, :])
        chunks.append(base[:, c * CH:(c + 1) * CH] - (acc + acc))
    dist = jnp.concatenate(chunks, axis=1)

    lane = lax.broadcasted_iota(jnp.int32, (1, N_PAD), 1)
    valid = lane < NUM_ENT
    dist = jnp.where(valid, dist, -jnp.inf)
    m = jnp.max(dist, axis=-1, keepdims=True)
    e = jnp.exp(dist - m)
    e = jnp.where(valid, e, 0.0)
    s = jnp.sum(e, axis=-1, keepdims=True)
    out_ref[...] = (e / s)[:, :NUM_ENT]


def kernel(e1, rel, X, A, ent_emb, rel_emb):
    del X, A
    e1 = e1.astype(jnp.int32)
    rel = rel.astype(jnp.int32)
    ent_pad = jnp.pad(ent_emb, ((0, 0), (0, D_PAD - DIM)))
    rel_pad = jnp.pad(rel_emb, ((0, 0), (0, D_PAD - DIM)))
    e1_rows, rel_rows = _sc_gather(ent_pad, rel_pad, e1, rel)
    ent_t = jnp.pad(ent_emb.T, ((0, 0), (0, N_PAD - NUM_ENT)))
    return pl.pallas_call(
        _score_body,
        grid=(B // B_BLK,),
        in_specs=[
            pl.BlockSpec((B_BLK, D_PAD), lambda i: (i, 0)),
            pl.BlockSpec((B_BLK, D_PAD), lambda i: (i, 0)),
            pl.BlockSpec((DIM, N_PAD), lambda i: (0, 0)),
        ],
        out_specs=pl.BlockSpec((B_BLK, NUM_ENT), lambda i: (i, 0)),
        out_shape=jax.ShapeDtypeStruct((B, NUM_ENT), jnp.float32),
    )(e1_rows, rel_rows, ent_t)
